# baseline (speedup 1.0000x reference)
import functools
import jax
import jax.numpy as jnp
from jax import lax
from jax.experimental import pallas as pl
from jax.experimental.pallas import tpu as pltpu
from jax.experimental.pallas import tpu_sc as plsc

NC = 2
NS = 16
NW = NC * NS
LANES = 16
BATCH = 16384
EMB = 64
PADW = 128
SUB = 16
BPW = BATCH // NW
CHUNK = 16
NCHUNK = BPW // CHUNK


def _body(user_hbm, movie_hbm, uemb_hbm, memb_hbm, ubias_hbm, mbias_hbm,
          gbias_hbm, out_hbm,
          uorig_v, morig_v, ubrow_v, mbrow_v, urows_v, mrows_v,
          ubrows_v, mbrows_v, gb_v, out_v, buf_v, sem, bsem):
    wid = lax.axis_index("s") * NC + lax.axis_index("c")
    base = wid * BPW

    for j in range(0, BPW, PADW):
        pltpu.sync_copy(user_hbm.at[pl.ds(base + j, PADW)],
                        uorig_v.at[pl.ds(j, PADW)])
        pltpu.sync_copy(movie_hbm.at[pl.ds(base + j, PADW)],
                        morig_v.at[pl.ds(j, PADW)])
    pltpu.sync_copy(gbias_hbm, gb_v.at[pl.ds(0, 1)])

    for j in range(0, BPW, LANES):
        sl = pl.ds(j, LANES)
        ubrow_v[sl] = lax.shift_right_logical(uorig_v[sl], 7)
        mbrow_v[sl] = lax.shift_right_logical(morig_v[sl], 7)

    gb = gb_v[pl.ds(0, LANES)][0]
    iota = jax.lax.iota(jnp.int32, LANES)
    base_idx = iota * (LANES + 1)

    def fire(c, slot):
        ebase = c * CHUNK
        uvec = uorig_v[pl.ds(ebase, LANES)]
        mvec = morig_v[pl.ds(ebase, LANES)]
        ub8 = lax.shift_left(lax.shift_right_logical(uvec, 4), 4)
        mb8 = lax.shift_left(lax.shift_right_logical(mvec, 4), 4)
        handles = [
            pltpu.async_copy(ubias_hbm.at[ubrow_v.at[pl.ds(ebase, CHUNK)]],
                             ubrows_v.at[pl.ds(slot * CHUNK, CHUNK)], bsem),
            pltpu.async_copy(mbias_hbm.at[mbrow_v.at[pl.ds(ebase, CHUNK)]],
                             mbrows_v.at[pl.ds(slot * CHUNK, CHUNK)], bsem),
        ]
        for j in range(LANES):
            ru = pl.multiple_of(ub8[j], SUB)
            rm = pl.multiple_of(mb8[j], SUB)
            handles.append(pltpu.async_copy(
                uemb_hbm.at[pl.ds(ru, SUB), :], urows_v.at[slot, j], sem))
            handles.append(pltpu.async_copy(
                memb_hbm.at[pl.ds(rm, SUB), :], mrows_v.at[slot, j], sem))
        return handles

    def compute(c, slot):
        ebase = c * CHUNK
        uvec = uorig_v[pl.ds(ebase, LANES)]
        mvec = morig_v[pl.ds(ebase, LANES)]
        usub = lax.bitwise_and(uvec, SUB - 1)
        msub = lax.bitwise_and(mvec, SUB - 1)
        for r in range(LANES):
            lu = usub[r]
            lm = msub[r]
            acc = None
            for d in range(0, EMB, 2 * LANES):
                u2 = urows_v[slot, r, lu, pl.ds(d, 2 * LANES)]
                m2 = mrows_v[slot, r, lm, pl.ds(d, 2 * LANES)]
                ua, ub = plsc.unpack(u2, format=plsc.PackFormat.INTERLEAVED)
                ma, mb = plsc.unpack(m2, format=plsc.PackFormat.INTERLEAVED)
                p = ua * ma + ub * mb
                acc = p if acc is None else acc + p
            buf_v[pl.ds(r * (LANES + 1), LANES)] = acc
        tot = None
        for col in range(LANES):
            v = plsc.load_gather(buf_v, [base_idx + col])
            tot = v if tot is None else tot + v
        ulane = lax.bitwise_and(uvec, 127)
        mlane = lax.bitwise_and(mvec, 127)
        bu = plsc.load_gather(ubrows_v, [iota + slot * CHUNK, ulane])
        bm = plsc.load_gather(mbrows_v, [iota + slot * CHUNK, mlane])
        out_v[pl.ds(ebase, LANES)] = tot + bu + bm + gb

    def pair_body(p, carry):
        c0 = p * 2
        c1 = c0 + 1
        h0 = fire(c0, 0)
        h1 = fire(c1, 1)
        for h in h0:
            h.wait()
        compute(c0, 0)
        for h in h1:
            h.wait()
        compute(c1, 1)
        return carry

    lax.fori_loop(0, NCHUNK // 2, pair_body, 0)

    pltpu.sync_copy(out_v, out_hbm.at[pl.ds(base, BPW)])


def kernel(user, movie, user_embedding, movie_embedding,
           user_bias_embedding, movie_bias_embedding, global_bias):
    n_user = user_embedding.shape[0]
    n_movie = movie_embedding.shape[0]
    ubr = -(-n_user // PADW)
    mbr = -(-n_movie // PADW)
    ubp = jnp.pad(user_bias_embedding,
                  ((0, ubr * PADW - n_user), (0, 0))).reshape(ubr, PADW)
    mbp = jnp.pad(movie_bias_embedding,
                  ((0, mbr * PADW - n_movie), (0, 0))).reshape(mbr, PADW)
    mesh = plsc.VectorSubcoreMesh(core_axis_name="c", subcore_axis_name="s",
                                  num_cores=NC, num_subcores=NS)
    run = pl.kernel(
        _body,
        out_type=jax.ShapeDtypeStruct((BATCH,), jnp.float32),
        mesh=mesh,
        compiler_params=pltpu.CompilerParams(needs_layout_passes=False,
                                             use_tc_tiling_on_sc=True),
        scratch_types=[
            pltpu.VMEM((BPW,), jnp.int32),
            pltpu.VMEM((BPW,), jnp.int32),
            pltpu.VMEM((BPW,), jnp.int32),
            pltpu.VMEM((BPW,), jnp.int32),
            pltpu.VMEM((2, CHUNK, SUB, EMB), jnp.bfloat16),
            pltpu.VMEM((2, CHUNK, SUB, EMB), jnp.bfloat16),
            pltpu.VMEM((2 * CHUNK, PADW), jnp.float32),
            pltpu.VMEM((2 * CHUNK, PADW), jnp.float32),
            pltpu.VMEM((LANES,), jnp.float32),
            pltpu.VMEM((BPW,), jnp.float32),
            pltpu.VMEM((LANES * (LANES + 1),), jnp.float32),
            pltpu.SemaphoreType.DMA,
            pltpu.SemaphoreType.DMA,
        ],
    )
    return run(user, movie, user_embedding.astype(jnp.bfloat16),
               movie_embedding.astype(jnp.bfloat16), ubp, mbp, global_bias)


# speedup vs baseline: 1.0228x; 1.0228x over previous
import functools
import jax
import jax.numpy as jnp
from jax import lax
from jax.experimental import pallas as pl
from jax.experimental.pallas import tpu as pltpu
from jax.experimental.pallas import tpu_sc as plsc

NC = 2
NS = 16
NW = NC * NS
LANES = 16
BATCH = 16384
EMB = 64
PADW = 128
SUB = 8
BPW = BATCH // NW
CHUNK = 16
NCHUNK = BPW // CHUNK


def _body(user_hbm, movie_hbm, uemb_hbm, memb_hbm, ubias_hbm, mbias_hbm,
          gbias_hbm, out_hbm,
          uorig_v, morig_v, ubrow_v, mbrow_v, urows_v, mrows_v,
          ubrows_v, mbrows_v, gb_v, out_v, buf_v, sem, bsem):
    wid = lax.axis_index("s") * NC + lax.axis_index("c")
    base = wid * BPW

    for j in range(0, BPW, PADW):
        pltpu.sync_copy(user_hbm.at[pl.ds(base + j, PADW)],
                        uorig_v.at[pl.ds(j, PADW)])
        pltpu.sync_copy(movie_hbm.at[pl.ds(base + j, PADW)],
                        morig_v.at[pl.ds(j, PADW)])
    pltpu.sync_copy(gbias_hbm, gb_v.at[pl.ds(0, 1)])

    for j in range(0, BPW, LANES):
        sl = pl.ds(j, LANES)
        ubrow_v[sl] = lax.shift_right_logical(uorig_v[sl], 7)
        mbrow_v[sl] = lax.shift_right_logical(morig_v[sl], 7)

    gb = gb_v[pl.ds(0, LANES)][0]
    iota = jax.lax.iota(jnp.int32, LANES)
    base_idx = iota * (LANES + 1)

    def fire(c, slot):
        ebase = c * CHUNK
        uvec = uorig_v[pl.ds(ebase, LANES)]
        mvec = morig_v[pl.ds(ebase, LANES)]
        ub8 = lax.shift_left(lax.shift_right_logical(uvec, 3), 3)
        mb8 = lax.shift_left(lax.shift_right_logical(mvec, 3), 3)
        handles = [
            pltpu.async_copy(ubias_hbm.at[ubrow_v.at[pl.ds(ebase, CHUNK)]],
                             ubrows_v.at[pl.ds(slot * CHUNK, CHUNK)], bsem),
            pltpu.async_copy(mbias_hbm.at[mbrow_v.at[pl.ds(ebase, CHUNK)]],
                             mbrows_v.at[pl.ds(slot * CHUNK, CHUNK)], bsem),
        ]
        for j in range(LANES):
            ru = pl.multiple_of(ub8[j], SUB)
            rm = pl.multiple_of(mb8[j], SUB)
            handles.append(pltpu.async_copy(
                uemb_hbm.at[pl.ds(ru, SUB), :], urows_v.at[slot, j], sem))
            handles.append(pltpu.async_copy(
                memb_hbm.at[pl.ds(rm, SUB), :], mrows_v.at[slot, j], sem))
        return handles

    def compute(c, slot):
        ebase = c * CHUNK
        uvec = uorig_v[pl.ds(ebase, LANES)]
        mvec = morig_v[pl.ds(ebase, LANES)]
        usub = lax.bitwise_and(uvec, SUB - 1)
        msub = lax.bitwise_and(mvec, SUB - 1)
        for r in range(LANES):
            lu = usub[r]
            lm = msub[r]
            acc = None
            for d in range(0, EMB, LANES):
                u = urows_v[slot, r, lu, pl.ds(d, LANES)]
                m = mrows_v[slot, r, lm, pl.ds(d, LANES)]
                p = u * m
                acc = p if acc is None else acc + p
            buf_v[pl.ds(r * (LANES + 1), LANES)] = acc
        tot = None
        for col in range(LANES):
            v = plsc.load_gather(buf_v, [base_idx + col])
            tot = v if tot is None else tot + v
        ulane = lax.bitwise_and(uvec, 127)
        mlane = lax.bitwise_and(mvec, 127)
        bu = plsc.load_gather(ubrows_v, [iota + slot * CHUNK, ulane])
        bm = plsc.load_gather(mbrows_v, [iota + slot * CHUNK, mlane])
        out_v[pl.ds(ebase, LANES)] = tot + bu + bm + gb

    def pair_body(p, carry):
        c0 = p * 2
        c1 = c0 + 1
        h0 = fire(c0, 0)
        h1 = fire(c1, 1)
        for h in h0:
            h.wait()
        compute(c0, 0)
        for h in h1:
            h.wait()
        compute(c1, 1)
        return carry

    lax.fori_loop(0, NCHUNK // 2, pair_body, 0)

    pltpu.sync_copy(out_v, out_hbm.at[pl.ds(base, BPW)])


def kernel(user, movie, user_embedding, movie_embedding,
           user_bias_embedding, movie_bias_embedding, global_bias):
    n_user = user_embedding.shape[0]
    n_movie = movie_embedding.shape[0]
    ubr = -(-n_user // PADW)
    mbr = -(-n_movie // PADW)
    ubp = jnp.pad(user_bias_embedding,
                  ((0, ubr * PADW - n_user), (0, 0))).reshape(ubr, PADW)
    mbp = jnp.pad(movie_bias_embedding,
                  ((0, mbr * PADW - n_movie), (0, 0))).reshape(mbr, PADW)
    mesh = plsc.VectorSubcoreMesh(core_axis_name="c", subcore_axis_name="s",
                                  num_cores=NC, num_subcores=NS)
    run = pl.kernel(
        _body,
        out_type=jax.ShapeDtypeStruct((BATCH,), jnp.float32),
        mesh=mesh,
        compiler_params=pltpu.CompilerParams(needs_layout_passes=False,
                                             use_tc_tiling_on_sc=True),
        scratch_types=[
            pltpu.VMEM((BPW,), jnp.int32),
            pltpu.VMEM((BPW,), jnp.int32),
            pltpu.VMEM((BPW,), jnp.int32),
            pltpu.VMEM((BPW,), jnp.int32),
            pltpu.VMEM((2, CHUNK, SUB, EMB), jnp.float32),
            pltpu.VMEM((2, CHUNK, SUB, EMB), jnp.float32),
            pltpu.VMEM((2 * CHUNK, PADW), jnp.float32),
            pltpu.VMEM((2 * CHUNK, PADW), jnp.float32),
            pltpu.VMEM((LANES,), jnp.float32),
            pltpu.VMEM((BPW,), jnp.float32),
            pltpu.VMEM((LANES * (LANES + 1),), jnp.float32),
            pltpu.SemaphoreType.DMA,
            pltpu.SemaphoreType.DMA,
        ],
    )
    return run(user, movie, user_embedding, movie_embedding, ubp, mbp,
               global_bias)
